# baseline (speedup 1.0000x reference)
import functools

import jax
import jax.numpy as jnp
from jax import lax
from jax.experimental import pallas as pl
from jax.experimental.pallas import tpu as pltpu
from jax.experimental.pallas import tpu_sc as plsc

N = 10000
D = 256
DH = 128
E = 160000
EP = 161792
CHUNK = 128
NCH = 79
PCH = 40
EPT_DEG = EP // 32
NR = 10112
RPT = NR // 16
HB = 10240
HPT = HB // 16
BLK = 5000
GRID = N // BLK


def _k1_body(x_ref, w1_ref, b1_ref, t_ref, stats_ref, acc_ref):
    i = pl.program_id(0)
    t = jnp.dot(x_ref[...], w1_ref[...], preferred_element_type=jnp.float32)
    t = t + b1_ref[...]
    t_ref[...] = t

    @pl.when(i == 0)
    def _():
        acc_ref[...] = jnp.zeros_like(acc_ref)

    acc_ref[0:1, :] += jnp.sum(t, axis=0, keepdims=True)
    acc_ref[1:2, :] += jnp.sum(t * t, axis=0, keepdims=True)

    @pl.when(i == GRID - 1)
    def _():
        stats_ref[...] = acc_ref[...]


def _k1(x, w1, b1r):
    return pl.pallas_call(
        _k1_body,
        grid=(GRID,),
        in_specs=[
            pl.BlockSpec((BLK, D), lambda i: (i, 0)),
            pl.BlockSpec((D, D), lambda i: (0, 0)),
            pl.BlockSpec((1, D), lambda i: (0, 0)),
        ],
        out_specs=[
            pl.BlockSpec((BLK, D), lambda i: (i, 0)),
            pl.BlockSpec((2, D), lambda i: (0, 0)),
        ],
        out_shape=[
            jax.ShapeDtypeStruct((N, D), jnp.float32),
            jax.ShapeDtypeStruct((2, D), jnp.float32),
        ],
        scratch_shapes=[pltpu.VMEM((2, D), jnp.float32)],
    )(x, w1, b1r)


def _k2_body(t_ref, stats_ref, gamma_ref, beta_ref, w2_ref, wg_ref, b2_ref,
             degp_ref, g2_ref, dinv_ref, wc_ref):
    i = pl.program_id(0)

    @pl.when(i == 0)
    def _():
        wc_ref[...] = jnp.dot(w2_ref[...], wg_ref[...],
                              preferred_element_type=jnp.float32)

    inv_n = 1.0 / N
    mean = stats_ref[0:1, :] * inv_n
    var = stats_ref[1:2, :] * inv_n - mean * mean
    a = gamma_ref[...] * lax.rsqrt(var + 1e-5)
    c = beta_ref[...] - mean * a
    r = jnp.maximum(t_ref[...] * a + c, 0.0)
    bc = jnp.dot(b2_ref[...], wg_ref[...], preferred_element_type=jnp.float32)
    hw = jnp.dot(r, wc_ref[...], preferred_element_type=jnp.float32) + bc
    deg = degp_ref[0, :, :] + degp_ref[1, :, :] + 1.0
    dinv = lax.rsqrt(deg)
    g = hw * dinv
    g2_ref[0, :, :] = g[:, :DH]
    g2_ref[1, :, :] = g[:, DH:]
    dinv_ref[...] = dinv


def _k2(t, stats, gamma_r, beta_r, w2, wg, b2r, degp):
    return pl.pallas_call(
        _k2_body,
        grid=(GRID,),
        in_specs=[
            pl.BlockSpec((BLK, D), lambda i: (i, 0)),
            pl.BlockSpec((2, D), lambda i: (0, 0)),
            pl.BlockSpec((1, D), lambda i: (0, 0)),
            pl.BlockSpec((1, D), lambda i: (0, 0)),
            pl.BlockSpec((D, D), lambda i: (0, 0)),
            pl.BlockSpec((D, D), lambda i: (0, 0)),
            pl.BlockSpec((1, D), lambda i: (0, 0)),
            pl.BlockSpec((2, BLK, 1), lambda i: (0, i, 0)),
        ],
        out_specs=[
            pl.BlockSpec((2, BLK, DH), lambda i: (0, i, 0)),
            pl.BlockSpec((BLK, 1), lambda i: (i, 0)),
        ],
        out_shape=[
            jax.ShapeDtypeStruct((2, N, DH), jnp.float32),
            jax.ShapeDtypeStruct((N, 1), jnp.float32),
        ],
        scratch_shapes=[pltpu.VMEM((D, D), jnp.float32)],
    )(t, stats, gamma_r, beta_r, w2, wg, b2r, degp)


def _k3_body(a_ref, g_ref, dinv_ref, bg_ref, out_ref):
    dinv = dinv_ref[...]
    left = (a_ref[0, :, :] + g_ref[0, :, :]) * dinv
    right = (a_ref[1, :, :] + g_ref[1, :, :]) * dinv
    out_ref[...] = jnp.concatenate([left, right], axis=1) + bg_ref[...]


def _k3(acc, g2, dinv2, bgr):
    return pl.pallas_call(
        _k3_body,
        grid=(GRID,),
        in_specs=[
            pl.BlockSpec((2, BLK, DH), lambda i: (0, i, 0)),
            pl.BlockSpec((2, BLK, DH), lambda i: (0, i, 0)),
            pl.BlockSpec((BLK, 1), lambda i: (i, 0)),
            pl.BlockSpec((1, D), lambda i: (0, 0)),
        ],
        out_specs=pl.BlockSpec((BLK, D), lambda i: (i, 0)),
        out_shape=jax.ShapeDtypeStruct((N, D), jnp.float32),
    )(acc, g2, dinv2, bgr)


def _sc_mesh():
    return plsc.VectorSubcoreMesh(core_axis_name="c", subcore_axis_name="s")


@functools.partial(
    pl.kernel,
    mesh=_sc_mesh(),
    compiler_params=pltpu.CompilerParams(needs_layout_passes=False),
    out_type=jax.ShapeDtypeStruct((2 * HB,), jnp.float32),
    scratch_types=[
        pltpu.VMEM((EPT_DEG,), jnp.int32),
        pltpu.VMEM((HB,), jnp.float32),
        pltpu.VMEM((HPT,), jnp.float32),
        pltpu.VMEM((HPT,), jnp.float32),
        pltpu.VMEM_SHARED((16, HB), jnp.float32),
    ],
)
def _s1_degree(cols_hbm, deg_hbm, cbuf, hist, acc, tmp, hist_sh):
    core = lax.axis_index("c")
    sid = lax.axis_index("s")
    wid = sid * 2 + core
    pltpu.sync_copy(cols_hbm.at[pl.ds(wid * EPT_DEG, EPT_DEG)], cbuf)

    zeros16 = jnp.zeros((16,), jnp.float32)
    ones16 = jnp.ones((16,), jnp.float32)

    def zh(i, carry):
        hist[pl.ds(i * 16, 16)] = zeros16
        return carry

    lax.fori_loop(0, HB // 16, zh, 0)

    def scat(i, carry):
        idx = cbuf[pl.ds(i * 16, 16)]
        plsc.addupdate_scatter(hist, [idx], ones16)
        return carry

    lax.fori_loop(0, EPT_DEG // 16, scat, 0)

    pltpu.sync_copy(hist, hist_sh.at[sid])
    plsc.subcore_barrier()

    def zacc(i, carry):
        acc[pl.ds(i * 16, 16)] = zeros16
        return carry

    lax.fori_loop(0, HPT // 16, zacc, 0)

    def red(t, carry):
        pltpu.sync_copy(hist_sh.at[t, pl.ds(sid * HPT, HPT)], tmp)

        def add(v, c2):
            acc[pl.ds(v * 16, 16)] += tmp[pl.ds(v * 16, 16)]
            return c2

        lax.fori_loop(0, HPT // 16, add, 0)
        return carry

    lax.fori_loop(0, 16, red, 0)
    pltpu.sync_copy(acc, deg_hbm.at[pl.ds(core * HB + sid * HPT, HPT)])


@functools.partial(
    pl.kernel,
    mesh=_sc_mesh(),
    compiler_params=pltpu.CompilerParams(needs_layout_passes=False),
    out_type=jax.ShapeDtypeStruct((2, NR, DH), jnp.float32),
    scratch_types=[
        pltpu.VMEM((PCH, CHUNK), jnp.int32),
        pltpu.VMEM((PCH, CHUNK), jnp.int32),
        pltpu.VMEM((2, CHUNK, DH), jnp.float32),
        pltpu.VMEM_SHARED((NR, DH), jnp.float32),
        pltpu.SemaphoreType.DMA,
    ],
)
def _s2_scatter(g2_hbm, rows_hbm, cols_hbm, out_hbm, ridx, cidx, rowbuf,
                a_sh, sem):
    core = lax.axis_index("c")
    sid = lax.axis_index("s")

    zeros16 = jnp.zeros((16,), jnp.float32)

    pltpu.async_copy(rows_hbm.at[sid, pl.ds(0, PCH)], ridx, sem)
    pltpu.async_copy(cols_hbm.at[sid, pl.ds(0, PCH)], cidx, sem)

    def zrow(i, carry):
        r = i // (DH // 16)
        v = i % (DH // 16)
        rowbuf[0, r, pl.ds(v * 16, 16)] = zeros16
        return carry

    lax.fori_loop(0, CHUNK * (DH // 16), zrow, 0)

    base = sid * RPT
    for k in range(RPT // CHUNK):
        pltpu.async_copy(rowbuf.at[0], a_sh.at[pl.ds(base + k * CHUNK, CHUNK)],
                         sem)
    rem = RPT % CHUNK
    if rem:
        pltpu.async_copy(rowbuf.at[0, pl.ds(0, rem)],
                         a_sh.at[pl.ds(base + (RPT // CHUNK) * CHUNK, rem)],
                         sem)
    pltpu.make_async_copy(rows_hbm.at[sid, pl.ds(0, PCH)], ridx, sem).wait()
    pltpu.make_async_copy(cols_hbm.at[sid, pl.ds(0, PCH)], cidx, sem).wait()
    for k in range(RPT // CHUNK):
        pltpu.make_async_copy(rowbuf.at[0],
                              a_sh.at[pl.ds(base + k * CHUNK, CHUNK)],
                              sem).wait()
    if rem:
        pltpu.make_async_copy(rowbuf.at[0, pl.ds(0, rem)],
                              a_sh.at[pl.ds(base + (RPT // CHUNK) * CHUNK,
                                            rem)], sem).wait()
    plsc.subcore_barrier()

    gsrc = g2_hbm.at[core]

    for p, nch in ((0, PCH), (1, NCH - PCH)):
        if p > 0:
            pltpu.sync_copy(rows_hbm.at[sid, pl.ds(p * PCH, PCH)], ridx)
            pltpu.sync_copy(cols_hbm.at[sid, pl.ds(p * PCH, PCH)], cidx)
        pltpu.async_copy(gsrc.at[ridx.at[0]], rowbuf.at[0], sem)

        def body(j, carry):
            b = lax.rem(j, 2)
            pltpu.make_async_copy(gsrc.at[ridx.at[j]], rowbuf.at[b],
                                  sem).wait()

            @pl.when(j + 1 < nch)
            def _():
                pltpu.async_copy(gsrc.at[ridx.at[j + 1]], rowbuf.at[1 - b],
                                 sem)

            pltpu.sync_copy(rowbuf.at[b], a_sh.at[cidx.at[j]], add=True)
            return carry

        lax.fori_loop(0, nch, body, 0)

    plsc.subcore_barrier()
    pltpu.sync_copy(a_sh.at[pl.ds(base, RPT)],
                    out_hbm.at[core, pl.ds(base, RPT)])


def kernel(x, edge_index, W1, b1, gamma, beta, W2, b2, Wg, bg):
    ei = edge_index.astype(jnp.int32)
    rows = ei[0]
    cols = ei[1]
    pad_dst = N + (jnp.arange(EP - E, dtype=jnp.int32) % (NR - N))
    rows_p = jnp.concatenate([rows, jnp.zeros((EP - E,), jnp.int32)])
    cols_p = jnp.concatenate([cols, pad_dst])
    dummy = jnp.zeros((16, 1, CHUNK), jnp.int32)
    rows3 = jnp.concatenate([rows_p.reshape(16, NCH, CHUNK), dummy], axis=1)
    cols3 = jnp.concatenate([cols_p.reshape(16, NCH, CHUNK), dummy], axis=1)

    t, stats = _k1(x, W1, b1.reshape(1, D))
    degp = _s1_degree(cols_p).reshape(2, HB)
    degp2 = degp[:, :N].reshape(2, N, 1)
    g2, dinv2 = _k2(t, stats, gamma.reshape(1, D), beta.reshape(1, D),
                    W2, Wg, b2.reshape(1, D), degp2)
    acc = _s2_scatter(g2, rows3, cols3)
    out = _k3(acc, g2, dinv2, bg.reshape(1, D))
    return out


# speedup vs baseline: 1.0030x; 1.0030x over previous
import functools

import jax
import jax.numpy as jnp
from jax import lax
from jax.experimental import pallas as pl
from jax.experimental.pallas import tpu as pltpu
from jax.experimental.pallas import tpu_sc as plsc

N = 10000
D = 256
DH = 128
E = 160000
EP = 161792
CHUNK = 128
NCH = 79
PCH = 40
EPT_DEG = EP // 32
NR = 10112
RPT = NR // 16
HB = 10240
HPT = HB // 16
BLK = 5000
GRID = N // BLK


def _k1_body(x_ref, w1_ref, b1_ref, t_ref, stats_ref, acc_ref):
    i = pl.program_id(0)
    t = jnp.dot(x_ref[...], w1_ref[...], preferred_element_type=jnp.float32)
    t = t + b1_ref[...]
    t_ref[...] = t

    @pl.when(i == 0)
    def _():
        acc_ref[...] = jnp.zeros_like(acc_ref)

    acc_ref[0:1, :] += jnp.sum(t, axis=0, keepdims=True)
    acc_ref[1:2, :] += jnp.sum(t * t, axis=0, keepdims=True)

    @pl.when(i == GRID - 1)
    def _():
        stats_ref[...] = acc_ref[...]


def _k1(x, w1, b1r):
    return pl.pallas_call(
        _k1_body,
        grid=(GRID,),
        in_specs=[
            pl.BlockSpec((BLK, D), lambda i: (i, 0)),
            pl.BlockSpec((D, D), lambda i: (0, 0)),
            pl.BlockSpec((1, D), lambda i: (0, 0)),
        ],
        out_specs=[
            pl.BlockSpec((BLK, D), lambda i: (i, 0)),
            pl.BlockSpec((2, D), lambda i: (0, 0)),
        ],
        out_shape=[
            jax.ShapeDtypeStruct((N, D), jnp.float32),
            jax.ShapeDtypeStruct((2, D), jnp.float32),
        ],
        scratch_shapes=[pltpu.VMEM((2, D), jnp.float32)],
    )(x, w1, b1r)


def _k2_body(t_ref, stats_ref, gamma_ref, beta_ref, w2_ref, wg_ref, b2_ref,
             degp_ref, g2_ref, dinv_ref, wc_ref):
    i = pl.program_id(0)

    @pl.when(i == 0)
    def _():
        wc_ref[...] = jnp.dot(w2_ref[...], wg_ref[...],
                              preferred_element_type=jnp.float32)

    inv_n = 1.0 / N
    mean = stats_ref[0:1, :] * inv_n
    var = stats_ref[1:2, :] * inv_n - mean * mean
    a = gamma_ref[...] * lax.rsqrt(var + 1e-5)
    c = beta_ref[...] - mean * a
    r = jnp.maximum(t_ref[...] * a + c, 0.0)
    bc = jnp.dot(b2_ref[...], wg_ref[...], preferred_element_type=jnp.float32)
    hw = jnp.dot(r, wc_ref[...], preferred_element_type=jnp.float32) + bc
    deg = degp_ref[0, :, :] + degp_ref[1, :, :] + 1.0
    dinv = lax.rsqrt(deg)
    g = hw * dinv
    g2_ref[0, :, :] = g[:, :DH]
    g2_ref[1, :, :] = g[:, DH:]
    dinv_ref[...] = dinv


def _k2(t, stats, gamma_r, beta_r, w2, wg, b2r, degp):
    return pl.pallas_call(
        _k2_body,
        grid=(GRID,),
        in_specs=[
            pl.BlockSpec((BLK, D), lambda i: (i, 0)),
            pl.BlockSpec((2, D), lambda i: (0, 0)),
            pl.BlockSpec((1, D), lambda i: (0, 0)),
            pl.BlockSpec((1, D), lambda i: (0, 0)),
            pl.BlockSpec((D, D), lambda i: (0, 0)),
            pl.BlockSpec((D, D), lambda i: (0, 0)),
            pl.BlockSpec((1, D), lambda i: (0, 0)),
            pl.BlockSpec((2, BLK, 1), lambda i: (0, i, 0)),
        ],
        out_specs=[
            pl.BlockSpec((2, BLK, DH), lambda i: (0, i, 0)),
            pl.BlockSpec((BLK, 1), lambda i: (i, 0)),
        ],
        out_shape=[
            jax.ShapeDtypeStruct((2, N, DH), jnp.float32),
            jax.ShapeDtypeStruct((N, 1), jnp.float32),
        ],
        scratch_shapes=[pltpu.VMEM((D, D), jnp.float32)],
    )(t, stats, gamma_r, beta_r, w2, wg, b2r, degp)


def _k3_body(a_ref, g_ref, dinv_ref, bg_ref, out_ref):
    dinv = dinv_ref[...]
    left = (a_ref[0, :, :] + g_ref[0, :, :]) * dinv
    right = (a_ref[1, :, :] + g_ref[1, :, :]) * dinv
    out_ref[...] = jnp.concatenate([left, right], axis=1) + bg_ref[...]


def _k3(acc, g2, dinv2, bgr):
    return pl.pallas_call(
        _k3_body,
        grid=(GRID,),
        in_specs=[
            pl.BlockSpec((2, BLK, DH), lambda i: (0, i, 0)),
            pl.BlockSpec((2, BLK, DH), lambda i: (0, i, 0)),
            pl.BlockSpec((BLK, 1), lambda i: (i, 0)),
            pl.BlockSpec((1, D), lambda i: (0, 0)),
        ],
        out_specs=pl.BlockSpec((BLK, D), lambda i: (i, 0)),
        out_shape=jax.ShapeDtypeStruct((N, D), jnp.float32),
    )(acc, g2, dinv2, bgr)


def _sc_mesh():
    return plsc.VectorSubcoreMesh(core_axis_name="c", subcore_axis_name="s")


@functools.partial(
    pl.kernel,
    mesh=_sc_mesh(),
    compiler_params=pltpu.CompilerParams(needs_layout_passes=False),
    out_type=jax.ShapeDtypeStruct((2 * HB,), jnp.float32),
    scratch_types=[
        pltpu.VMEM((EPT_DEG,), jnp.int32),
        pltpu.VMEM((HB,), jnp.float32),
        pltpu.VMEM((HPT,), jnp.float32),
        pltpu.VMEM((HPT,), jnp.float32),
        pltpu.VMEM_SHARED((16, HB), jnp.float32),
    ],
)
def _s1_degree(cols_hbm, deg_hbm, cbuf, hist, acc, tmp, hist_sh):
    core = lax.axis_index("c")
    sid = lax.axis_index("s")
    wid = sid * 2 + core
    pltpu.sync_copy(cols_hbm.at[pl.ds(wid * EPT_DEG, EPT_DEG)], cbuf)

    zeros16 = jnp.zeros((16,), jnp.float32)
    ones16 = jnp.ones((16,), jnp.float32)

    def zh(i, carry):
        for u in range(4):
            hist[pl.ds(i * 64 + u * 16, 16)] = zeros16
        return carry

    lax.fori_loop(0, HB // 64, zh, 0)

    def scat(i, carry):
        for u in range(4):
            idx = cbuf[pl.ds(i * 64 + u * 16, 16)]
            plsc.addupdate_scatter(hist, [idx], ones16)
        return carry

    lax.fori_loop(0, EPT_DEG // 64, scat, 0)

    pltpu.sync_copy(hist, hist_sh.at[sid])
    plsc.subcore_barrier()

    def zacc(i, carry):
        acc[pl.ds(i * 16, 16)] = zeros16
        return carry

    lax.fori_loop(0, HPT // 16, zacc, 0)

    def red(t, carry):
        pltpu.sync_copy(hist_sh.at[t, pl.ds(sid * HPT, HPT)], tmp)

        def add(v, c2):
            for u in range(4):
                sl = pl.ds(v * 64 + u * 16, 16)
                acc[sl] += tmp[sl]
            return c2

        lax.fori_loop(0, HPT // 64, add, 0)
        return carry

    lax.fori_loop(0, 16, red, 0)
    pltpu.sync_copy(acc, deg_hbm.at[pl.ds(core * HB + sid * HPT, HPT)])


@functools.partial(
    pl.kernel,
    mesh=_sc_mesh(),
    compiler_params=pltpu.CompilerParams(needs_layout_passes=False),
    out_type=jax.ShapeDtypeStruct((2, NR, DH), jnp.float32),
    scratch_types=[
        pltpu.VMEM((PCH, CHUNK), jnp.int32),
        pltpu.VMEM((PCH, CHUNK), jnp.int32),
        pltpu.VMEM((2, CHUNK, DH), jnp.float32),
        pltpu.VMEM_SHARED((NR, DH), jnp.float32),
        pltpu.SemaphoreType.DMA,
    ],
)
def _s2_scatter(g2_hbm, rows_hbm, cols_hbm, out_hbm, ridx, cidx, rowbuf,
                a_sh, sem):
    core = lax.axis_index("c")
    sid = lax.axis_index("s")

    zeros16 = jnp.zeros((16,), jnp.float32)

    pltpu.async_copy(rows_hbm.at[sid, pl.ds(0, PCH)], ridx, sem)
    pltpu.async_copy(cols_hbm.at[sid, pl.ds(0, PCH)], cidx, sem)

    def zrow(i, carry):
        r = i // (DH // 16)
        v = i % (DH // 16)
        rowbuf[0, r, pl.ds(v * 16, 16)] = zeros16
        return carry

    lax.fori_loop(0, CHUNK * (DH // 16), zrow, 0)

    base = sid * RPT
    for k in range(RPT // CHUNK):
        pltpu.async_copy(rowbuf.at[0], a_sh.at[pl.ds(base + k * CHUNK, CHUNK)],
                         sem)
    rem = RPT % CHUNK
    if rem:
        pltpu.async_copy(rowbuf.at[0, pl.ds(0, rem)],
                         a_sh.at[pl.ds(base + (RPT // CHUNK) * CHUNK, rem)],
                         sem)
    pltpu.make_async_copy(rows_hbm.at[sid, pl.ds(0, PCH)], ridx, sem).wait()
    pltpu.make_async_copy(cols_hbm.at[sid, pl.ds(0, PCH)], cidx, sem).wait()
    for k in range(RPT // CHUNK):
        pltpu.make_async_copy(rowbuf.at[0],
                              a_sh.at[pl.ds(base + k * CHUNK, CHUNK)],
                              sem).wait()
    if rem:
        pltpu.make_async_copy(rowbuf.at[0, pl.ds(0, rem)],
                              a_sh.at[pl.ds(base + (RPT // CHUNK) * CHUNK,
                                            rem)], sem).wait()
    plsc.subcore_barrier()

    gsrc = g2_hbm.at[core]

    for p, nch in ((0, PCH), (1, NCH - PCH)):
        if p > 0:
            pltpu.sync_copy(rows_hbm.at[sid, pl.ds(p * PCH, PCH)], ridx)
            pltpu.sync_copy(cols_hbm.at[sid, pl.ds(p * PCH, PCH)], cidx)
        pltpu.async_copy(gsrc.at[ridx.at[0]], rowbuf.at[0], sem)

        def body(j, carry):
            b = lax.rem(j, 2)
            pltpu.make_async_copy(gsrc.at[ridx.at[j]], rowbuf.at[b],
                                  sem).wait()

            @pl.when(j + 1 < nch)
            def _():
                pltpu.async_copy(gsrc.at[ridx.at[j + 1]], rowbuf.at[1 - b],
                                 sem)

            pltpu.sync_copy(rowbuf.at[b], a_sh.at[cidx.at[j]], add=True)
            return carry

        lax.fori_loop(0, nch, body, 0)

    plsc.subcore_barrier()
    pltpu.sync_copy(a_sh.at[pl.ds(base, RPT)],
                    out_hbm.at[core, pl.ds(base, RPT)])


def kernel(x, edge_index, W1, b1, gamma, beta, W2, b2, Wg, bg):
    ei = edge_index.astype(jnp.int32)
    rows = ei[0]
    cols = ei[1]
    pad_dst = N + (jnp.arange(EP - E, dtype=jnp.int32) % (NR - N))
    rows_p = jnp.concatenate([rows, jnp.zeros((EP - E,), jnp.int32)])
    cols_p = jnp.concatenate([cols, pad_dst])
    dummy = jnp.zeros((16, 1, CHUNK), jnp.int32)
    rows3 = jnp.concatenate([rows_p.reshape(16, NCH, CHUNK), dummy], axis=1)
    cols3 = jnp.concatenate([cols_p.reshape(16, NCH, CHUNK), dummy], axis=1)

    t, stats = _k1(x, W1, b1.reshape(1, D))
    degp = _s1_degree(cols_p).reshape(2, HB)
    degp2 = degp[:, :N].reshape(2, N, 1)
    g2, dinv2 = _k2(t, stats, gamma.reshape(1, D), beta.reshape(1, D),
                    W2, Wg, b2.reshape(1, D), degp2)
    acc = _s2_scatter(g2, rows3, cols3)
    out = _k3(acc, g2, dinv2, bg.reshape(1, D))
    return out
